# Initial kernel scaffold; baseline (speedup 1.0000x reference)
import jax
import jax.numpy as jnp
from jax.experimental import pallas as pl


def kernel(g_i, mention_scores, start_indices, end_indices, coarse_W, dist_tab, W1, b1, W2, b2, Wf_W, Wf_b):
    raise NotImplementedError("write your pallas kernel here")


# speedup vs baseline: 2.2246x; 2.2246x over previous
import functools

import jax
import jax.numpy as jnp
from jax import lax
from jax.experimental import pallas as pl
from jax.experimental.pallas import tpu as pltpu
from jax.experimental.pallas import tpu_sc as plsc

K_TOP = 50
NEG_MASK = -1e9
NEG_OUT = -1e10
SENT = -1e30


_BM1 = 256


def _k1_body(g_blk, gT, cw, ms_blk, msT, sc_ref, bi_ref, ant_ref):
    k = gT.shape[1]
    bm = g_blk.shape[0]
    pid = pl.program_id(0)
    c = jnp.dot(g_blk[...], cw[...], preferred_element_type=jnp.float32)
    ant = jnp.dot(c, gT[...], preferred_element_type=jnp.float32)
    ant = ant + ms_blk[...] + msT[...]
    rows = pid * bm + lax.broadcasted_iota(jnp.int32, (bm, k), 0)
    cols = lax.broadcasted_iota(jnp.int32, (bm, k), 1)
    ant_ref[...] = jnp.where(cols < rows, ant, NEG_MASK)
    lane64 = lax.broadcasted_iota(jnp.int32, (bm, 64), 1)

    def body(t, carry):
        bs, bi = carry
        a = ant_ref[...]
        m = jnp.max(a, axis=1, keepdims=True)
        idx = jnp.min(jnp.where(a == m, cols, k), axis=1, keepdims=True)
        ant_ref[...] = jnp.where(cols == idx, SENT, a)
        bs = jnp.where(lane64 == t, m, bs)
        bi = jnp.where(lane64 == t, idx, bi)
        return bs, bi

    bs0 = jnp.zeros((bm, 64), jnp.float32)
    bi0 = jnp.zeros((bm, 64), jnp.int32)
    bs, bi = lax.fori_loop(0, K_TOP, body, (bs0, bi0))
    rows64 = pid * bm + lax.broadcasted_iota(jnp.int32, (bm, 64), 0)
    valid = (lane64 < rows64) & (lane64 < K_TOP)
    sc_ref[...] = jnp.where(valid, bs, 0.0)
    bi_ref[...] = bi


def _coarse_topk(g, gT, cw, ms, msT):
    k, d = g.shape
    grid = k // _BM1
    return pl.pallas_call(
        _k1_body,
        grid=(grid,),
        in_specs=[
            pl.BlockSpec((_BM1, d), lambda i: (i, 0)),
            pl.BlockSpec((d, k), lambda i: (0, 0)),
            pl.BlockSpec((d, d), lambda i: (0, 0)),
            pl.BlockSpec((_BM1, 1), lambda i: (i, 0)),
            pl.BlockSpec((1, k), lambda i: (0, 0)),
        ],
        out_specs=[
            pl.BlockSpec((_BM1, 64), lambda i: (i, 0)),
            pl.BlockSpec((_BM1, 64), lambda i: (i, 0)),
        ],
        out_shape=[
            jax.ShapeDtypeStruct((k, 64), jnp.float32),
            jax.ShapeDtypeStruct((k, 64), jnp.int32),
        ],
        scratch_shapes=[pltpu.VMEM((_BM1, k), jnp.float32)],
    )(g, gT, cw, ms, msT)


def _sc_gather(table, flat_idx, start_pad=None):
    n = flat_idx.shape[0]
    d = table.shape[1]
    info = plsc.get_sparse_core_info()
    nc, ns = info.num_cores, info.num_subcores
    nw = nc * ns
    per_w = n // nw
    chunk = 200
    n_chunks = per_w // chunk
    mesh = plsc.VectorSubcoreMesh(core_axis_name="c", subcore_axis_name="s")
    with_start = start_pad is not None

    out_type = jax.ShapeDtypeStruct((n, d), jnp.float32)
    scratch = [
        pltpu.VMEM((chunk,), jnp.int32),
        pltpu.VMEM((chunk, d), jnp.float32),
        pltpu.SemaphoreType.DMA,
    ]
    if with_start:
        ds = start_pad.shape[1]
        out_type = [out_type, jax.ShapeDtypeStruct((n, ds), jnp.int32)]
        scratch += [pltpu.VMEM((chunk, ds), jnp.int32), pltpu.SemaphoreType.DMA]

    @functools.partial(pl.kernel, mesh=mesh, out_type=out_type, scratch_types=scratch)
    def gathered(*refs):
        if with_start:
            (tab_hbm, sp_hbm, idx_hbm, jg_hbm, sj_hbm,
             idx_v, rows_v, sem1, srows_v, sem2) = refs
        else:
            tab_hbm, idx_hbm, jg_hbm, idx_v, rows_v, sem1 = refs
        wid = lax.axis_index("s") * nc + lax.axis_index("c")
        base0 = wid * per_w

        def chunk_body(ci, _):
            base = base0 + ci * chunk
            pltpu.sync_copy(idx_hbm.at[pl.ds(base, chunk)], idx_v)
            cp1 = pltpu.async_copy(tab_hbm.at[idx_v], rows_v, sem1)
            if with_start:
                cp2 = pltpu.async_copy(sp_hbm.at[idx_v], srows_v, sem2)
            cp1.wait()
            pltpu.sync_copy(rows_v, jg_hbm.at[pl.ds(base, chunk)])
            if with_start:
                cp2.wait()
                pltpu.sync_copy(srows_v, sj_hbm.at[pl.ds(base, chunk)])
            return 0

        lax.fori_loop(0, n_chunks, chunk_body, 0)

    if with_start:
        return gathered(table, start_pad, flat_idx)
    return gathered(table, flat_idx)


_BM3 = 16


def _k3_body(g_blk, jg_blk, sj_blk, end_blk, w1, dtab, w1d, b1, w2, b2, bins_row, s_ref):
    npair = jg_blk.shape[0]
    pr = lax.broadcasted_iota(jnp.int32, (npair, 16), 0) // K_TOP
    c16 = lax.broadcasted_iota(jnp.int32, (npair, 16), 1)
    rep = (pr == c16).astype(jnp.float32)
    g_exp = jnp.dot(rep, g_blk[...], preferred_element_type=jnp.float32)
    jg = jg_blk[...]
    x = jnp.concatenate([g_exp, jg, g_exp * jg], axis=1).astype(jnp.bfloat16)
    acc = jnp.dot(x, w1[...], preferred_element_type=jnp.float32)
    endf = jnp.dot(rep, end_blk[...].astype(jnp.float32), preferred_element_type=jnp.float32)
    dist = endf - sj_blk[...][:, :1].astype(jnp.float32)
    binsv = jnp.sum((dist > bins_row[...]).astype(jnp.float32), axis=1, keepdims=True)
    oh = ((binsv == c16.astype(jnp.float32)) & (c16 < 12)).astype(jnp.float32)
    phiproj = jnp.dot(dtab[...], w1d[...], preferred_element_type=jnp.float32)
    acc = acc + jnp.dot(oh, phiproj, preferred_element_type=jnp.float32) + b1[...]
    h = jnp.where(acc >= 0, acc, 0.01 * acc)
    s_ref[...] = jnp.dot(h, w2[...], preferred_element_type=jnp.float32) + b2[...]


def _pair_mlp(g, jg, sj, end_col, w1_bf, dt_pad, w1d, b1r, w2c, b2r, bins_row):
    k, d = g.shape
    n = jg.shape[0]
    grid = k // _BM3
    bp = _BM3 * K_TOP
    h1 = w1_bf.shape[1]
    return pl.pallas_call(
        _k3_body,
        grid=(grid,),
        in_specs=[
            pl.BlockSpec((_BM3, d), lambda i: (i, 0)),
            pl.BlockSpec((bp, d), lambda i: (i, 0)),
            pl.BlockSpec((bp, sj.shape[1]), lambda i: (i, 0)),
            pl.BlockSpec((_BM3, 1), lambda i: (i, 0)),
            pl.BlockSpec((3 * d, h1), lambda i: (0, 0)),
            pl.BlockSpec((16, 20), lambda i: (0, 0)),
            pl.BlockSpec((20, h1), lambda i: (0, 0)),
            pl.BlockSpec((1, h1), lambda i: (0, 0)),
            pl.BlockSpec((h1, 1), lambda i: (0, 0)),
            pl.BlockSpec((1, 1), lambda i: (0, 0)),
            pl.BlockSpec((1, 16), lambda i: (0, 0)),
        ],
        out_specs=pl.BlockSpec((bp, 1), lambda i: (i, 0)),
        out_shape=jax.ShapeDtypeStruct((n, 1), jnp.float32),
    )(g, jg, sj, end_col, w1_bf, dt_pad, w1d, b1r, w2c, b2r, bins_row)


_BM4 = 32


def _k4_body(g_blk, jg3_blk, sa_blk, sc_blk, wf1, wf2, wfb, gout):
    bm = g_blk.shape[0]
    pid = pl.program_id(0)
    rows = pid * bm + lax.broadcasted_iota(jnp.int32, (bm, 64), 0)
    lane = lax.broadcasted_iota(jnp.int32, (bm, 64), 1)
    valid = (lane < rows) & (lane < K_TOP)
    coref = sa_blk[...] + sc_blk[...]
    logits = jnp.where(valid, coref, NEG_OUT)
    m = jnp.maximum(jnp.max(logits, axis=1, keepdims=True), 0.0)
    e = jnp.where(valid, jnp.exp(logits - m), 0.0)
    e0 = jnp.exp(-m)
    den = e0 + jnp.sum(e, axis=1, keepdims=True)
    p = e / den
    p0 = e0 / den
    p3 = jnp.reshape(p, (bm, 1, 64))[:, :, :K_TOP]
    an = lax.dot_general(
        p3, jg3_blk[...],
        dimension_numbers=(((2,), (1,)), ((0,), (0,))),
        preferred_element_type=jnp.float32,
    )
    g = g_blk[...]
    an = jnp.reshape(an, (bm, g.shape[1])) + p0 * g
    x = (jnp.dot(g, wf1[...], preferred_element_type=jnp.float32)
         + jnp.dot(an, wf2[...], preferred_element_type=jnp.float32) + wfb[...])
    f = 1.0 / (1.0 + jnp.exp(-x))
    gout[...] = f * g + (1.0 - f) * an


def _refine(g, jg3, sa64, sc64, wf1, wf2, wfb):
    k, d = g.shape
    grid = k // _BM4
    return pl.pallas_call(
        _k4_body,
        grid=(grid,),
        in_specs=[
            pl.BlockSpec((_BM4, d), lambda i: (i, 0)),
            pl.BlockSpec((_BM4, K_TOP, d), lambda i: (i, 0, 0)),
            pl.BlockSpec((_BM4, 64), lambda i: (i, 0)),
            pl.BlockSpec((_BM4, 64), lambda i: (i, 0)),
            pl.BlockSpec((d, d), lambda i: (0, 0)),
            pl.BlockSpec((d, d), lambda i: (0, 0)),
            pl.BlockSpec((1, d), lambda i: (0, 0)),
        ],
        out_specs=pl.BlockSpec((_BM4, d), lambda i: (i, 0)),
        out_shape=jax.ShapeDtypeStruct((k, d), jnp.float32),
    )(g, jg3, sa64, sc64, wf1, wf2, wfb)


def kernel(g_i, mention_scores, start_indices, end_indices, coarse_W, dist_tab,
           W1, b1, W2, b2, Wf_W, Wf_b):
    k, d = g_i.shape
    n = k * K_TOP
    h1 = W1.shape[1]

    gT = g_i.T
    ms = mention_scores
    msT = mention_scores.reshape(1, k)

    sc64, bi64 = _coarse_topk(g_i, gT, coarse_W, ms, msT)
    flat_idx = bi64[:, :K_TOP].reshape(n)
    start_pad = jnp.broadcast_to(start_indices[:, None], (k, 128)).astype(jnp.int32)
    end_col = end_indices.reshape(k, 1).astype(jnp.int32)

    w1_bf = W1[: 3 * d].astype(jnp.bfloat16)
    w1d = W1[3 * d:]
    dt_pad = jnp.pad(dist_tab, ((0, 16 - dist_tab.shape[0]), (0, 0)))
    b1r = b1.reshape(1, h1)
    w2c = W2.reshape(h1, 1)
    b2r = b2.reshape(1, 1)
    bins_row = jnp.concatenate(
        [jnp.array([1, 2, 3, 4, 8, 16, 32, 64, 128, 256, 384], jnp.float32),
         jnp.full((5,), 1e9, jnp.float32)]).reshape(1, 16)
    wf1 = Wf_W[:d]
    wf2 = Wf_W[d:]
    wfb = Wf_b.reshape(1, d)

    jg1, sj1 = _sc_gather(g_i, flat_idx, start_pad)
    sa1 = _pair_mlp(g_i, jg1, sj1, end_col, w1_bf, dt_pad, w1d, b1r, w2c, b2r, bins_row)
    sa1_64 = jnp.pad(sa1.reshape(k, K_TOP), ((0, 0), (0, 14)))
    g2 = _refine(g_i, jg1.reshape(k, K_TOP, d), sa1_64, sc64, wf1, wf2, wfb)

    jg2 = _sc_gather(g2, flat_idx)
    sa2 = _pair_mlp(g2, jg2, sj1, end_col, w1_bf, dt_pad, w1d, b1r, w2c, b2r, bins_row)

    coref = sa2.reshape(k, K_TOP) + sc64[:, :K_TOP]
    pos_mask = jnp.arange(K_TOP)[None, :] < jnp.arange(k)[:, None]
    scores = jnp.concatenate(
        [jnp.zeros((k, 1), jnp.float32), jnp.where(pos_mask, coref, NEG_OUT)], axis=1)
    return scores
